# Initial kernel scaffold; baseline (speedup 1.0000x reference)
import jax
import jax.numpy as jnp
from jax.experimental import pallas as pl


def kernel(x, ih_vals, bias_ih, hh_vals, bias_hh, ih_rows, ih_cols, hh_rows, hh_cols):
    raise NotImplementedError("write your pallas kernel here")


# speedup vs baseline: 4.5509x; 4.5509x over previous
import functools

import jax
import jax.numpy as jnp
from jax import lax
from jax.experimental import pallas as pl
from jax.experimental.pallas import tpu as pltpu
from jax.experimental.pallas import tpu_sc as plsc

H = 16384
F_IN = 16384
B = 64
S = 16
NNZ = 268435

NCORE = 2
NSUBC = 16
NW = NCORE * NSUBC
CH = 128
NCH = -(-NNZ // (NW * CH))
NNZ_PAD = NW * NCH * CH

ROWS_PER_W = H // NW
ROWS_PER_S = H // NSUBC

_mesh = plsc.VectorSubcoreMesh(core_axis_name="c", subcore_axis_name="s")


def _bcast16(i):
    return jnp.full((16,), i, dtype=jnp.int32)


def _spmm_body(x_t, h, ih_cols, ih_rows, ih_vals, hh_cols, hh_rows, hh_vals,
               part_out, acc, zbuf, gbuf, cols_v, rows_v, vals_v, gsem):
    c = lax.axis_index("c")
    s = lax.axis_index("s")
    w = c * NSUBC + s

    zero = jnp.zeros((16,), jnp.float32)

    def _zb(i, _):
        for j in range(B // 16):
            zbuf[i, pl.ds(j * 16, 16)] = zero
        return 0

    lax.fori_loop(0, CH, _zb, 0, unroll=4)
    for i in range(ROWS_PER_S // CH):
        pltpu.sync_copy(zbuf, acc.at[pl.ds(s * ROWS_PER_S + i * CH, CH)])
    plsc.subcore_barrier()

    def _run_matrix(table, cols3, rows3, vals3):
        def _chunk(k, _):
            pltpu.sync_copy(cols3.at[w, k], cols_v)
            pltpu.sync_copy(rows3.at[w, k], rows_v)
            pltpu.sync_copy(vals3.at[w, k], vals_v)
            pltpu.async_copy(table.at[cols_v], gbuf, gsem).wait()

            def _scale(i, _):
                bv = plsc.load_gather(vals_v, [_bcast16(i)])
                for j in range(B // 16):
                    gbuf[i, pl.ds(j * 16, 16)] = gbuf[i, pl.ds(j * 16, 16)] * bv
                return 0

            lax.fori_loop(0, CH, _scale, 0, unroll=8)
            pltpu.sync_copy(gbuf, acc.at[rows_v], add=True)
            return 0

        lax.fori_loop(0, NCH, _chunk, 0)

    _run_matrix(x_t, ih_cols, ih_rows, ih_vals)
    _run_matrix(h, hh_cols, hh_rows, hh_vals)

    plsc.subcore_barrier()
    for i in range(ROWS_PER_S // CH):
        r = s * ROWS_PER_S + i * CH
        pltpu.sync_copy(acc.at[pl.ds(r, CH)], part_out.at[c, pl.ds(r, CH)])


_params = pltpu.CompilerParams(needs_layout_passes=False,
                               use_tc_tiling_on_sc=False)

_k1 = pl.kernel(
    _spmm_body,
    out_type=jax.ShapeDtypeStruct((NCORE, H, B), jnp.float32),
    mesh=_mesh,
    compiler_params=_params,
    scratch_types=[
        pltpu.VMEM_SHARED((H, B), jnp.float32),
        pltpu.VMEM((CH, B), jnp.float32),
        pltpu.VMEM((CH, B), jnp.float32),
        pltpu.VMEM((CH,), jnp.int32),
        pltpu.VMEM((CH,), jnp.int32),
        pltpu.VMEM((CH,), jnp.float32),
        pltpu.SemaphoreType.DMA,
    ],
)


def _tanh_body(part, bias, h_out, p0v, p1v, bv, hbuf):
    c = lax.axis_index("c")
    s = lax.axis_index("s")
    w = c * NSUBC + s
    r0 = w * ROWS_PER_W

    pltpu.sync_copy(part.at[0, pl.ds(r0, ROWS_PER_W)], p0v)
    pltpu.sync_copy(part.at[1, pl.ds(r0, ROWS_PER_W)], p1v)
    pltpu.sync_copy(bias.at[pl.ds(r0, ROWS_PER_W)], bv)

    one = jnp.full((16,), 1.0, jnp.float32)
    two = jnp.full((16,), 2.0, jnp.float32)

    def _row(i, _):
        bb = plsc.load_gather(bv, [_bcast16(i)])
        for j in range(B // 16):
            z = p0v[i, pl.ds(j * 16, 16)] + p1v[i, pl.ds(j * 16, 16)] + bb
            e = jnp.exp(z * two)
            hbuf[i, pl.ds(j * 16, 16)] = one - two / (e + one)
        return 0

    lax.fori_loop(0, ROWS_PER_W, _row, 0, unroll=4)
    pltpu.sync_copy(hbuf, h_out.at[pl.ds(r0, ROWS_PER_W)])


_k2 = pl.kernel(
    _tanh_body,
    out_type=jax.ShapeDtypeStruct((H, B), jnp.float32),
    mesh=_mesh,
    compiler_params=_params,
    scratch_types=[
        pltpu.VMEM((ROWS_PER_W, B), jnp.float32),
        pltpu.VMEM((ROWS_PER_W, B), jnp.float32),
        pltpu.VMEM((ROWS_PER_W,), jnp.float32),
        pltpu.VMEM((ROWS_PER_W, B), jnp.float32),
    ],
)


def _prep(rows, cols, vals):
    pad = NNZ_PAD - NNZ
    rows = jnp.pad(rows, (0, pad)).reshape(NW, NCH, CH)
    cols = jnp.pad(cols, (0, pad)).reshape(NW, NCH, CH)
    vals = jnp.pad(vals, (0, pad)).reshape(NW, NCH, CH)
    return rows, cols, vals


def kernel(x, ih_vals, bias_ih, hh_vals, bias_hh, ih_rows, ih_cols, hh_rows, hh_cols):
    xp = jnp.transpose(x, (1, 2, 0))
    ihr, ihc, ihv = _prep(ih_rows, ih_cols, ih_vals)
    hhr, hhc, hhv = _prep(hh_rows, hh_cols, hh_vals)
    bias = (bias_ih + bias_hh).reshape(H)

    h = jnp.zeros((H, B), jnp.float32)
    outs = []
    for t in range(S):
        part = _k1(xp[t], h, ihc, ihr, ihv, hhc, hhr, hhv)
        h = _k2(part, bias)
        outs.append(h)

    out = jnp.transpose(jnp.stack(outs), (2, 0, 1))
    h_final = jnp.transpose(h[None, :, :], (2, 0, 1))
    return (out, h_final)


# speedup vs baseline: 6.0857x; 1.3372x over previous
import jax
import jax.numpy as jnp
from jax import lax
from jax.experimental import pallas as pl
from jax.experimental.pallas import tpu as pltpu
from jax.experimental.pallas import tpu_sc as plsc

H = 16384
F_IN = 16384
B = 64
S = 16
NNZ = 268435

NCORE = 2
NSUBC = 16
NW = NCORE * NSUBC
CH = 128
NCH = -(-NNZ // (NW * CH))
NNZ_PAD = NW * NCH * CH

ROWS_PER_W = H // NW
ROWS_PER_S = H // NSUBC

_mesh = plsc.VectorSubcoreMesh(core_axis_name="c", subcore_axis_name="s")


def _bcast16(i):
    return jnp.full((16,), i, dtype=jnp.int32)


def _spmm_body(x_t, h, ih_cols, ih_rows, ih_vals, hh_cols, hh_rows, hh_vals,
               part_out, acc, gb0, gb1, cA, rA, vA, gsem0, gsem1):
    c = lax.axis_index("c")
    s = lax.axis_index("s")
    w = c * NSUBC + s

    zero = jnp.zeros((16,), jnp.float32)

    def _zb(i, _):
        for j in range(B // 16):
            gb0[i, pl.ds(j * 16, 16)] = zero
        return 0

    lax.fori_loop(0, CH, _zb, 0, unroll=8)
    for i in range(ROWS_PER_S // CH):
        pltpu.sync_copy(gb0, acc.at[pl.ds(s * ROWS_PER_S + i * CH, CH)])
    plsc.subcore_barrier()

    gbufs = (gb0, gb1)
    gsems = (gsem0, gsem1)

    def _run_matrix(table, cols3, rows3, vals3):
        pltpu.sync_copy(cols3.at[w], cA)
        pltpu.sync_copy(rows3.at[w], rA)
        pltpu.sync_copy(vals3.at[w], vA)

        def _scale(gb, k):
            def _grp(g, _):
                for l in range(16):
                    i = g * 16 + l
                    bv = plsc.load_gather(vA, [_bcast16(k), _bcast16(i)])
                    for j in range(B // 16):
                        gb[i, pl.ds(j * 16, 16)] = gb[i, pl.ds(j * 16, 16)] * bv
                return 0

            lax.fori_loop(0, CH // 16, _grp, 0)

        pltpu.async_copy(table.at[cA.at[0]], gb0, gsem0)

        def _iter(ko, _):
            for b in range(2):
                k = ko * 2 + b
                gb, gsem = gbufs[b], gsems[b]
                ob, osem = gbufs[1 - b], gsems[1 - b]
                pltpu.make_async_copy(table.at[cA.at[k]], gb, gsem).wait()
                if b == 0:
                    pltpu.async_copy(table.at[cA.at[k + 1]], ob, osem)
                else:
                    @pl.when(ko < NCH // 2 - 1)
                    def _():
                        pltpu.async_copy(table.at[cA.at[k + 1]], ob, osem)
                _scale(gb, k)
                pltpu.sync_copy(gb, acc.at[rA.at[k]], add=True)
            return 0

        lax.fori_loop(0, NCH // 2, _iter, 0)

    _run_matrix(x_t, ih_cols, ih_rows, ih_vals)
    _run_matrix(h, hh_cols, hh_rows, hh_vals)

    plsc.subcore_barrier()
    for i in range(ROWS_PER_S // CH):
        r = s * ROWS_PER_S + i * CH
        pltpu.sync_copy(acc.at[pl.ds(r, CH)], part_out.at[c, pl.ds(r, CH)])


_params = pltpu.CompilerParams(needs_layout_passes=False,
                               use_tc_tiling_on_sc=False)

_k1 = pl.kernel(
    _spmm_body,
    out_type=jax.ShapeDtypeStruct((NCORE, H, B), jnp.float32),
    mesh=_mesh,
    compiler_params=_params,
    scratch_types=[
        pltpu.VMEM_SHARED((H, B), jnp.float32),
        pltpu.VMEM((CH, B), jnp.float32),
        pltpu.VMEM((CH, B), jnp.float32),
        pltpu.VMEM((NCH, CH), jnp.int32),
        pltpu.VMEM((NCH, CH), jnp.int32),
        pltpu.VMEM((NCH, CH), jnp.float32),
        pltpu.SemaphoreType.DMA,
        pltpu.SemaphoreType.DMA,
    ],
)


def _tanh_tc_body(p0, p1, b_ih, b_hh, h_out):
    h_out[...] = jnp.tanh(p0[...] + p1[...] + b_ih[...] + b_hh[...])


_k2 = pl.pallas_call(
    _tanh_tc_body,
    grid=(NW,),
    in_specs=[
        pl.BlockSpec((ROWS_PER_W, B), lambda i: (i, 0)),
        pl.BlockSpec((ROWS_PER_W, B), lambda i: (i, 0)),
        pl.BlockSpec((ROWS_PER_W, 1), lambda i: (i, 0)),
        pl.BlockSpec((ROWS_PER_W, 1), lambda i: (i, 0)),
    ],
    out_specs=pl.BlockSpec((ROWS_PER_W, B), lambda i: (i, 0)),
    out_shape=jax.ShapeDtypeStruct((H, B), jnp.float32),
)


def _prep(rows, cols, vals):
    pad = NNZ_PAD - NNZ
    rows = jnp.pad(rows, (0, pad)).reshape(NW, NCH, CH)
    cols = jnp.pad(cols, (0, pad)).reshape(NW, NCH, CH)
    vals = jnp.pad(vals, (0, pad)).reshape(NW, NCH, CH)
    return rows, cols, vals


def kernel(x, ih_vals, bias_ih, hh_vals, bias_hh, ih_rows, ih_cols, hh_rows, hh_cols):
    xp = jnp.transpose(x, (1, 2, 0))
    ihr, ihc, ihv = _prep(ih_rows, ih_cols, ih_vals)
    hhr, hhc, hhv = _prep(hh_rows, hh_cols, hh_vals)

    h = jnp.zeros((H, B), jnp.float32)
    outs = []
    for t in range(S):
        part = _k1(xp[t], h, ihc, ihr, ihv, hhc, hhr, hhv)
        h = _k2(part[0], part[1], bias_ih, bias_hh)
        outs.append(h)

    out = jnp.transpose(jnp.stack(outs), (2, 0, 1))
    h_final = jnp.transpose(h[None, :, :], (2, 0, 1))
    return (out, h_final)


# speedup vs baseline: 8.6718x; 1.4250x over previous
import jax
import jax.numpy as jnp
from jax import lax
from jax.experimental import pallas as pl
from jax.experimental.pallas import tpu as pltpu
from jax.experimental.pallas import tpu_sc as plsc

H = 16384
F_IN = 16384
B = 64
S = 16
NNZ = 268435

NCORE = 2
NSUBC = 16
NW = NCORE * NSUBC
CH = 128
NCH = -(-NNZ // (NW * CH))
NNZ_PAD = NW * NCH * CH

ROWS_PER_W = H // NW
ROWS_PER_S = H // NSUBC

_mesh = plsc.VectorSubcoreMesh(core_axis_name="c", subcore_axis_name="s")


def _bcast16(i):
    return jnp.full((16,), i, dtype=jnp.int32)


def _spmm_body(x_t, h, ih_cols, ih_rows, ih_vals, hh_cols, hh_rows, hh_vals,
               part_out, acc, gb0, gb1, sbuf, cA, rA, vA, gsem0, gsem1):
    c = lax.axis_index("c")
    s = lax.axis_index("s")
    w = c * NSUBC + s

    zero = jnp.zeros((16,), jnp.float32)

    def _zb(i, _):
        for j in range(B // 16):
            gb0[i, pl.ds(j * 16, 16)] = zero
        return 0

    lax.fori_loop(0, CH, _zb, 0, unroll=8)
    for i in range(ROWS_PER_S // CH):
        pltpu.sync_copy(gb0, acc.at[pl.ds(s * ROWS_PER_S + i * CH, CH)])
    plsc.subcore_barrier()

    gbufs = (gb0, gb1)
    gsems = (gsem0, gsem1)

    def _run_matrix(table, cols3, rows3, vals3):
        pltpu.sync_copy(cols3.at[w], cA)
        pltpu.sync_copy(rows3.at[w], rA)
        pltpu.sync_copy(vals3.at[w], vA)

        def _scale(gb, k):
            def _grp(g, _):
                for l in range(16):
                    i = g * 16 + l
                    bv = plsc.load_gather(vA, [_bcast16(k), _bcast16(i)])
                    a = [gb[i, pl.ds(j * 16, 16)] for j in range(B // 16)]
                    for j in range(B // 16):
                        sbuf[i, pl.ds(j * 16, 16)] = a[j] * bv
                return 0

            lax.fori_loop(0, CH // 16, _grp, 0)

        pltpu.async_copy(table.at[cA.at[0]], gb0, gsem0)

        def _iter(ko, _):
            for b in range(2):
                k = ko * 2 + b
                gb, gsem = gbufs[b], gsems[b]
                ob, osem = gbufs[1 - b], gsems[1 - b]
                pltpu.make_async_copy(table.at[cA.at[k]], gb, gsem).wait()
                if b == 0:
                    pltpu.async_copy(table.at[cA.at[k + 1]], ob, osem)
                else:
                    @pl.when(ko < NCH // 2 - 1)
                    def _():
                        pltpu.async_copy(table.at[cA.at[k + 1]], ob, osem)
                _scale(gb, k)
                pltpu.sync_copy(sbuf, acc.at[rA.at[k]], add=True)
            return 0

        lax.fori_loop(0, NCH // 2, _iter, 0)

    _run_matrix(x_t, ih_cols, ih_rows, ih_vals)
    _run_matrix(h, hh_cols, hh_rows, hh_vals)

    plsc.subcore_barrier()
    for i in range(ROWS_PER_S // CH):
        r = s * ROWS_PER_S + i * CH
        pltpu.sync_copy(acc.at[pl.ds(r, CH)], part_out.at[c, pl.ds(r, CH)])


_params = pltpu.CompilerParams(needs_layout_passes=False,
                               use_tc_tiling_on_sc=False)

_k1 = pl.kernel(
    _spmm_body,
    out_type=jax.ShapeDtypeStruct((NCORE, H, B), jnp.float32),
    mesh=_mesh,
    compiler_params=_params,
    scratch_types=[
        pltpu.VMEM_SHARED((H, B), jnp.float32),
        pltpu.VMEM((CH, B), jnp.float32),
        pltpu.VMEM((CH, B), jnp.float32),
        pltpu.VMEM((CH, B), jnp.float32),
        pltpu.VMEM((NCH, CH), jnp.int32),
        pltpu.VMEM((NCH, CH), jnp.int32),
        pltpu.VMEM((NCH, CH), jnp.float32),
        pltpu.SemaphoreType.DMA,
        pltpu.SemaphoreType.DMA,
    ],
)


def _tanh_tc_body(p0, p1, b_ih, b_hh, h_out):
    h_out[...] = jnp.tanh(p0[...] + p1[...] + b_ih[...] + b_hh[...])


_k2 = pl.pallas_call(
    _tanh_tc_body,
    grid=(NW,),
    in_specs=[
        pl.BlockSpec((ROWS_PER_W, B), lambda i: (i, 0)),
        pl.BlockSpec((ROWS_PER_W, B), lambda i: (i, 0)),
        pl.BlockSpec((ROWS_PER_W, 1), lambda i: (i, 0)),
        pl.BlockSpec((ROWS_PER_W, 1), lambda i: (i, 0)),
    ],
    out_specs=pl.BlockSpec((ROWS_PER_W, B), lambda i: (i, 0)),
    out_shape=jax.ShapeDtypeStruct((H, B), jnp.float32),
)


def _prep(rows, cols, vals):
    pad = NNZ_PAD - NNZ
    rows = jnp.pad(rows, (0, pad)).reshape(NW, NCH, CH)
    cols = jnp.pad(cols, (0, pad)).reshape(NW, NCH, CH)
    vals = jnp.pad(vals, (0, pad)).reshape(NW, NCH, CH)
    return rows, cols, vals


def kernel(x, ih_vals, bias_ih, hh_vals, bias_hh, ih_rows, ih_cols, hh_rows, hh_cols):
    xp = jnp.transpose(x, (1, 2, 0))
    ihr, ihc, ihv = _prep(ih_rows, ih_cols, ih_vals)
    hhr, hhc, hhv = _prep(hh_rows, hh_cols, hh_vals)

    h = jnp.zeros((H, B), jnp.float32)
    outs = []
    for t in range(S):
        part = _k1(xp[t], h, ihc, ihr, ihv, hhc, hhr, hhv)
        h = _k2(part[0], part[1], bias_ih, bias_hh)
        outs.append(h)

    out = jnp.transpose(jnp.stack(outs), (2, 0, 1))
    h_final = jnp.transpose(h[None, :, :], (2, 0, 1))
    return (out, h_final)
